# baseline (speedup 1.0000x reference)
import dataclasses
import functools

import jax
import jax.numpy as jnp
from jax import lax
from jax.experimental import pallas as pl
from jax.experimental.pallas import tpu as pltpu
from jax.experimental.pallas import tpu_sc as plsc

N_NODES = 10000
N_EDGES = 320000
D_FEAT = 128

NC = 2
NS = 16
NW = NC * NS
L = 16

E_PER_TILE = N_EDGES // NW
W = 80
NWIN = E_PER_TILE // W
GROUPS = W // L


def _sc_body(x_hbm, src_hbm, dst_hbm, dist_hbm, dir_hbm,
             idx_s, idx_d, a_ref, b_ref, dist_v):
    wid = lax.axis_index("s") * NC + lax.axis_index("c")
    tile_base = wid * E_PER_TILE
    lanes = lax.broadcasted_iota(jnp.int32, (L,), 0)

    @pl.loop(0, NWIN)
    def _window(w):
        base = tile_base + w * W
        pltpu.sync_copy(src_hbm.at[pl.ds(base, W)], idx_s)
        pltpu.sync_copy(dst_hbm.at[pl.ds(base, W)], idx_d)
        pltpu.sync_copy(x_hbm.at[idx_s], a_ref)
        pltpu.sync_copy(x_hbm.at[idx_d], b_ref)

        @pl.loop(0, GROUPS)
        def _group(g):
            rows = g * L + lanes

            def _sq_step(j, acc):
                col = jnp.full((L,), j, jnp.int32)
                av = plsc.load_gather(a_ref, [rows, col])
                bv = plsc.load_gather(b_ref, [rows, col])
                r = av - bv
                plsc.store_scatter(a_ref, [rows, col], r)
                return acc + r * r

            s = lax.fori_loop(0, D_FEAT, _sq_step, jnp.zeros((L,), jnp.float32))

            i = plsc.bitcast(s, jnp.int32)
            i = jnp.int32(0x5F3759DF) - lax.shift_right_logical(i, 1)
            y = plsc.bitcast(i, jnp.float32)
            half_s = s * 0.5
            for _ in range(3):
                y = y * (1.5 - half_s * y * y)
            dist = s * y
            dist = jnp.where(s > 0.0, dist, 0.0)
            dist_v[pl.ds(g * L, L)] = dist
            inv = 1.0 / (1.0 + dist)

            def _scale_step(j, carry):
                col = jnp.full((L,), j, jnp.int32)
                r = plsc.load_gather(a_ref, [rows, col])
                plsc.store_scatter(a_ref, [rows, col], r * inv)
                return carry

            lax.fori_loop(0, D_FEAT, _scale_step, jnp.int32(0))

        pltpu.sync_copy(dist_v, dist_hbm.at[pl.ds(base, W)])
        pltpu.sync_copy(a_ref, dir_hbm.at[pl.ds(base, W)])


@jax.jit
def kernel(x, edge_index):
    src = edge_index[0].astype(jnp.int32)
    dst = edge_index[1].astype(jnp.int32)

    mesh = plsc.VectorSubcoreMesh(core_axis_name="c", subcore_axis_name="s")
    cp = pltpu.CompilerParams()
    if "needs_layout_passes" in pltpu.CompilerParams.__dataclass_fields__:
        cp = dataclasses.replace(cp, needs_layout_passes=False)
    sc_kernel = pl.kernel(
        _sc_body,
        compiler_params=cp,
        out_type=(
            jax.ShapeDtypeStruct((N_EDGES,), jnp.float32),
            jax.ShapeDtypeStruct((N_EDGES, D_FEAT), jnp.float32),
        ),
        mesh=mesh,
        scratch_types=[
            pltpu.VMEM((W,), jnp.int32),
            pltpu.VMEM((W,), jnp.int32),
            pltpu.VMEM((W, D_FEAT), jnp.float32),
            pltpu.VMEM((W, D_FEAT), jnp.float32),
            pltpu.VMEM((W,), jnp.float32),
        ],
    )
    edge_dist, edge_dir = sc_kernel(x, src, dst)
    return edge_dist, edge_dir


# speedup vs baseline: 1.6135x; 1.6135x over previous
import dataclasses
import functools

import jax
import jax.numpy as jnp
from jax import lax
from jax.experimental import pallas as pl
from jax.experimental.pallas import tpu as pltpu
from jax.experimental.pallas import tpu_sc as plsc

N_NODES = 10000
N_EDGES = 320000
D_FEAT = 128

NC = 2
NS = 16
NW = NC * NS
L = 16

E_PER_TILE = N_EDGES // NW
W = 80
NWIN = E_PER_TILE // W
GROUPS = W // L


def _sc_body(x_hbm, src_hbm, dst_hbm, dist_hbm, dir_hbm,
             idx_s, idx_d, a_ref, b_ref, dist_v, sem_in, sem_out):
    wid = lax.axis_index("s") * NC + lax.axis_index("c")
    tile_base = wid * E_PER_TILE
    lanes = lax.broadcasted_iota(jnp.int32, (L,), 0)

    pltpu.sync_copy(src_hbm.at[pl.ds(tile_base, E_PER_TILE)], idx_s)
    pltpu.sync_copy(dst_hbm.at[pl.ds(tile_base, E_PER_TILE)], idx_d)

    def _gather_start(w, off):
        pltpu.async_copy(x_hbm.at[idx_s.at[pl.ds(w * W, W)]],
                         a_ref.at[pl.ds(off, W)], sem_in)
        pltpu.async_copy(x_hbm.at[idx_d.at[pl.ds(w * W, W)]],
                         b_ref.at[pl.ds(off, W)], sem_in)

    def _compute_window(off):
        @pl.loop(0, GROUPS)
        def _group(g):
            rows = off + g * L + lanes

            cols = [jnp.full((L,), j, jnp.int32) for j in range(D_FEAT)]
            accs = [jnp.zeros((L,), jnp.float32) for _ in range(4)]
            DEPTH = 4
            avq = [plsc.load_gather(a_ref, [rows, cols[j]])
                   for j in range(DEPTH)]
            bvq = [plsc.load_gather(b_ref, [rows, cols[j]])
                   for j in range(DEPTH)]
            for j in range(D_FEAT):
                if j + DEPTH < D_FEAT:
                    avq.append(plsc.load_gather(a_ref, [rows, cols[j + DEPTH]]))
                    bvq.append(plsc.load_gather(b_ref, [rows, cols[j + DEPTH]]))
                r = avq[j] - bvq[j]
                plsc.store_scatter(a_ref, [rows, cols[j]], r)
                accs[j % 4] = accs[j % 4] + r * r
            s = (accs[0] + accs[1]) + (accs[2] + accs[3])

            i = plsc.bitcast(s, jnp.int32)
            i = jnp.int32(0x5F3759DF) - lax.shift_right_logical(i, 1)
            y = plsc.bitcast(i, jnp.float32)
            half_s = s * 0.5
            for _ in range(3):
                y = y * (1.5 - half_s * y * y)
            dist = s * y
            dist = jnp.where(s > 0.0, dist, 0.0)
            dist_v[pl.ds(off + g * L, L)] = dist
            inv = 1.0 / (1.0 + dist)

            rq = [plsc.load_gather(a_ref, [rows, cols[j]])
                  for j in range(DEPTH)]
            for j in range(D_FEAT):
                if j + DEPTH < D_FEAT:
                    rq.append(plsc.load_gather(a_ref, [rows, cols[j + DEPTH]]))
                plsc.store_scatter(a_ref, [rows, cols[j]], rq[j] * inv)

    _gather_start(0, 0)

    @pl.loop(0, NWIN)
    def _window(w):
        off = (w & 1) * W
        offn = W - off
        base = tile_base + w * W

        pltpu.make_async_copy(x_hbm.at[pl.ds(0, W)],
                              a_ref.at[pl.ds(off, W)], sem_in).wait()
        pltpu.make_async_copy(x_hbm.at[pl.ds(0, W)],
                              b_ref.at[pl.ds(off, W)], sem_in).wait()

        @pl.when(w >= 1)
        def _retire():
            pltpu.make_async_copy(dist_v.at[pl.ds(offn, W)],
                                  dist_hbm.at[pl.ds(tile_base, W)],
                                  sem_out).wait()
            pltpu.make_async_copy(a_ref.at[pl.ds(offn, W)],
                                  dir_hbm.at[pl.ds(tile_base, W)],
                                  sem_out).wait()

        @pl.when(w + 1 < NWIN)
        def _prefetch():
            _gather_start(w + 1, offn)

        _compute_window(off)

        pltpu.async_copy(dist_v.at[pl.ds(off, W)],
                         dist_hbm.at[pl.ds(base, W)], sem_out)
        pltpu.async_copy(a_ref.at[pl.ds(off, W)],
                         dir_hbm.at[pl.ds(base, W)], sem_out)

    fin = ((NWIN - 1) & 1) * W
    pltpu.make_async_copy(dist_v.at[pl.ds(fin, W)],
                          dist_hbm.at[pl.ds(tile_base, W)], sem_out).wait()
    pltpu.make_async_copy(a_ref.at[pl.ds(fin, W)],
                          dir_hbm.at[pl.ds(tile_base, W)], sem_out).wait()


@jax.jit
def kernel(x, edge_index):
    src = edge_index[0].astype(jnp.int32)
    dst = edge_index[1].astype(jnp.int32)

    mesh = plsc.VectorSubcoreMesh(core_axis_name="c", subcore_axis_name="s")
    cp = pltpu.CompilerParams()
    if "needs_layout_passes" in pltpu.CompilerParams.__dataclass_fields__:
        cp = dataclasses.replace(cp, needs_layout_passes=False)
    sc_kernel = pl.kernel(
        _sc_body,
        compiler_params=cp,
        out_type=(
            jax.ShapeDtypeStruct((N_EDGES,), jnp.float32),
            jax.ShapeDtypeStruct((N_EDGES, D_FEAT), jnp.float32),
        ),
        mesh=mesh,
        scratch_types=[
            pltpu.VMEM((E_PER_TILE,), jnp.int32),
            pltpu.VMEM((E_PER_TILE,), jnp.int32),
            pltpu.VMEM((2 * W, D_FEAT), jnp.float32),
            pltpu.VMEM((2 * W, D_FEAT), jnp.float32),
            pltpu.VMEM((2 * W,), jnp.float32),
            pltpu.SemaphoreType.DMA,
            pltpu.SemaphoreType.DMA,
        ],
    )
    edge_dist, edge_dir = sc_kernel(x, src, dst)
    return edge_dist, edge_dir
